# baseline (speedup 1.0000x reference)
import functools

import jax
import jax.numpy as jnp
from jax import lax
from jax.experimental import pallas as pl
from jax.experimental.pallas import tpu as pltpu
from jax.experimental.pallas import tpu_sc as plsc

BATCH = 16384
HIDDEN = 128
CHUNK = 64
HBM_TAIL_CHUNKS = 2


@functools.cache
def _build_gather():
    info = plsc.get_sparse_core_info()
    num_workers = info.num_cores * info.num_subcores
    b_per_w = BATCH // num_workers
    n_chunks = b_per_w // CHUNK

    mesh = plsc.VectorSubcoreMesh(core_axis_name="c", subcore_axis_name="s")

    @functools.partial(
        pl.kernel,
        mesh=mesh,
        out_type=jax.ShapeDtypeStruct((BATCH, HIDDEN), jnp.float32),
        scratch_types=[
            pltpu.VMEM((n_chunks, CHUNK), jnp.int32),
            pltpu.VMEM((b_per_w, HIDDEN), jnp.float32),
            pltpu.VMEM_SHARED((1001, HIDDEN), jnp.float32),
            pltpu.SemaphoreType.DMA,
            pltpu.SemaphoreType.DMA,
            pltpu.SemaphoreType.DMA,
        ],
    )
    def gather_kernel(labels_hbm, table_hbm, out_hbm, idx_v, rows_v, tbl_s, gsem, ssem, hsem):
        sid = lax.axis_index("s")
        wid = sid * info.num_cores + lax.axis_index("c")
        base = wid * b_per_w
        pltpu.sync_copy(labels_hbm.at[wid], idx_v)
        for t, (lo, n) in enumerate(((0, 256), (256, 256), (512, 256), (768, 233))):
            @pl.when(sid == t)
            def _stage(lo=lo, n=n):
                pltpu.sync_copy(
                    table_hbm.at[pl.ds(lo, n)],
                    tbl_s.at[pl.ds(lo, n)],
                )
        plsc.subcore_barrier()
        n_spmem = n_chunks - HBM_TAIL_CHUNKS
        gathers = [
            pltpu.async_copy(
                tbl_s.at[idx_v.at[j]],
                rows_v.at[pl.ds(j * CHUNK, CHUNK)],
                gsem,
            )
            for j in range(n_spmem)
        ] + [
            pltpu.async_copy(
                table_hbm.at[idx_v.at[j]],
                rows_v.at[pl.ds(j * CHUNK, CHUNK)],
                hsem,
            )
            for j in range(n_spmem, n_chunks)
        ]
        scatters = []
        for j in range(n_chunks):
            gathers[j].wait()
            scatters.append(
                pltpu.async_copy(
                    rows_v.at[pl.ds(j * CHUNK, CHUNK)],
                    out_hbm.at[pl.ds(base + j * CHUNK, CHUNK)],
                    ssem,
                )
            )
        for s in scatters:
            s.wait()

    return gather_kernel, num_workers, n_chunks


def kernel(labels, embedding_table, train=False):
    del train
    gather_kernel, num_workers, n_chunks = _build_gather()
    labels3 = labels.astype(jnp.int32).reshape(num_workers, n_chunks, CHUNK)
    return gather_kernel(labels3, embedding_table)


# speedup vs baseline: 1.0597x; 1.0597x over previous
import functools

import jax
import jax.numpy as jnp
from jax import lax
from jax.experimental import pallas as pl
from jax.experimental.pallas import tpu as pltpu
from jax.experimental.pallas import tpu_sc as plsc

BATCH = 16384
HIDDEN = 128
CHUNK = 32
HBM_TAIL_CHUNKS = 0


@functools.cache
def _build_gather():
    info = plsc.get_sparse_core_info()
    num_workers = info.num_cores * info.num_subcores
    b_per_w = BATCH // num_workers
    n_chunks = b_per_w // CHUNK

    mesh = plsc.VectorSubcoreMesh(core_axis_name="c", subcore_axis_name="s")

    @functools.partial(
        pl.kernel,
        mesh=mesh,
        out_type=jax.ShapeDtypeStruct((BATCH, HIDDEN), jnp.float32),
        scratch_types=[
            pltpu.VMEM((n_chunks, CHUNK), jnp.int32),
            pltpu.VMEM((b_per_w, HIDDEN), jnp.float32),
            pltpu.VMEM_SHARED((1001, HIDDEN), jnp.float32),
            pltpu.SemaphoreType.DMA,
            pltpu.SemaphoreType.DMA,
            pltpu.SemaphoreType.DMA,
        ],
    )
    def gather_kernel(labels_hbm, table_hbm, out_hbm, idx_v, rows_v, tbl_s, gsem, ssem, hsem):
        sid = lax.axis_index("s")
        wid = sid * info.num_cores + lax.axis_index("c")
        base = wid * b_per_w
        pltpu.sync_copy(labels_hbm.at[wid], idx_v)
        for t, (lo, n) in enumerate(((0, 256), (256, 256), (512, 256), (768, 233))):
            @pl.when(sid == t)
            def _stage(lo=lo, n=n):
                pltpu.sync_copy(
                    table_hbm.at[pl.ds(lo, n)],
                    tbl_s.at[pl.ds(lo, n)],
                )
        plsc.subcore_barrier()
        n_spmem = n_chunks - HBM_TAIL_CHUNKS
        gathers = [
            pltpu.async_copy(
                tbl_s.at[idx_v.at[j]],
                rows_v.at[pl.ds(j * CHUNK, CHUNK)],
                gsem,
            )
            for j in range(n_spmem)
        ] + [
            pltpu.async_copy(
                table_hbm.at[idx_v.at[j]],
                rows_v.at[pl.ds(j * CHUNK, CHUNK)],
                hsem,
            )
            for j in range(n_spmem, n_chunks)
        ]
        scatters = []
        for j in range(n_chunks):
            gathers[j].wait()
            scatters.append(
                pltpu.async_copy(
                    rows_v.at[pl.ds(j * CHUNK, CHUNK)],
                    out_hbm.at[pl.ds(base + j * CHUNK, CHUNK)],
                    ssem,
                )
            )
        for s in scatters:
            s.wait()

    return gather_kernel, num_workers, n_chunks


def kernel(labels, embedding_table, train=False):
    del train
    gather_kernel, num_workers, n_chunks = _build_gather()
    labels3 = labels.astype(jnp.int32).reshape(num_workers, n_chunks, CHUNK)
    return gather_kernel(labels3, embedding_table)


# speedup vs baseline: 1.0696x; 1.0093x over previous
import functools

import jax
import jax.numpy as jnp
from jax import lax
from jax.experimental import pallas as pl
from jax.experimental.pallas import tpu as pltpu
from jax.experimental.pallas import tpu_sc as plsc

BATCH = 16384
HIDDEN = 128
CHUNK = 64


@functools.cache
def _build_gather():
    info = plsc.get_sparse_core_info()
    num_workers = info.num_cores * info.num_subcores
    b_per_w = BATCH // num_workers
    n_chunks = b_per_w // CHUNK

    mesh = plsc.VectorSubcoreMesh(core_axis_name="c", subcore_axis_name="s")

    @functools.partial(
        pl.kernel,
        mesh=mesh,
        out_type=jax.ShapeDtypeStruct((BATCH, HIDDEN), jnp.float32),
        scratch_types=[
            pltpu.VMEM((n_chunks, CHUNK), jnp.int32),
            pltpu.VMEM((b_per_w, HIDDEN), jnp.float32),
            pltpu.VMEM_SHARED((1001, HIDDEN), jnp.float32),
            pltpu.SemaphoreType.DMA,
            pltpu.SemaphoreType.DMA,
        ],
    )
    def gather_kernel(labels_hbm, table_hbm, out_hbm, idx_v, rows_v, tbl_s, gsem, ssem):
        sid = lax.axis_index("s")
        wid = sid * info.num_cores + lax.axis_index("c")
        base = wid * b_per_w
        pltpu.sync_copy(labels_hbm.at[wid], idx_v)
        for t, (lo, n) in enumerate(((0, 256), (256, 256), (512, 256), (768, 233))):
            @pl.when(sid == t)
            def _stage(lo=lo, n=n):
                pltpu.sync_copy(
                    table_hbm.at[pl.ds(lo, n)],
                    tbl_s.at[pl.ds(lo, n)],
                )
        plsc.subcore_barrier()
        gathers = [
            pltpu.async_copy(
                tbl_s.at[idx_v.at[j]],
                rows_v.at[pl.ds(j * CHUNK, CHUNK)],
                gsem,
            )
            for j in range(n_chunks)
        ]
        scatters = []
        for j in range(n_chunks):
            gathers[j].wait()
            scatters.append(
                pltpu.async_copy(
                    rows_v.at[pl.ds(j * CHUNK, CHUNK)],
                    out_hbm.at[pl.ds(base + j * CHUNK, CHUNK)],
                    ssem,
                )
            )
        for s in scatters:
            s.wait()

    return gather_kernel, num_workers, n_chunks


def kernel(labels, embedding_table, train=False):
    del train
    gather_kernel, num_workers, n_chunks = _build_gather()
    labels3 = labels.astype(jnp.int32).reshape(num_workers, n_chunks, CHUNK)
    return gather_kernel(labels3, embedding_table)
